# baseline (speedup 1.0000x reference)
import functools

import jax
import jax.numpy as jnp
from jax import lax
from jax.experimental import pallas as pl
from jax.experimental.pallas import tpu as pltpu
from jax.experimental.pallas import tpu_sc as plsc

PRE_SEQ_LEN = 128
OUT_DIM = 12 * 2 * 768
BATCH = 32
NUM_ROWS = BATCH * PRE_SEQ_LEN

NC = 2
NS = 16
COLS = OUT_DIM // NC
ROWS_PER_TILE = NUM_ROWS // NS
CHUNK = 4
TAB_ROWS_PER_TILE = PRE_SEQ_LEN // NS


def _body(idx_hbm, table_hbm, out_hbm, spmem, idx_v, buf, sem):
    c = lax.axis_index("c")
    s = lax.axis_index("s")
    col0 = c * COLS

    tr0 = s * TAB_ROWS_PER_TILE
    pltpu.sync_copy(
        table_hbm.at[pl.ds(tr0, TAB_ROWS_PER_TILE), pl.ds(col0, COLS)],
        spmem.at[pl.ds(tr0, TAB_ROWS_PER_TILE), :],
    )
    plsc.subcore_barrier()

    base = s * ROWS_PER_TILE
    pltpu.sync_copy(
        idx_hbm.at[pl.ds(base // CHUNK, ROWS_PER_TILE // CHUNK)], idx_v
    )

    def step(i, carry):
        off = i * CHUNK
        pltpu.async_copy(spmem.at[idx_v.at[i]], buf, sem).wait()
        pltpu.sync_copy(
            buf, out_hbm.at[pl.ds(base + off, CHUNK), pl.ds(col0, COLS)]
        )
        return carry

    lax.fori_loop(0, ROWS_PER_TILE // CHUNK, step, 0)


_gather = functools.partial(
    pl.kernel,
    out_type=jax.ShapeDtypeStruct((NUM_ROWS, OUT_DIM), jnp.float32),
    mesh=plsc.VectorSubcoreMesh(core_axis_name="c", subcore_axis_name="s"),
    scratch_types=[
        pltpu.VMEM_SHARED((PRE_SEQ_LEN, COLS), jnp.float32),
        pltpu.VMEM((ROWS_PER_TILE // CHUNK, CHUNK), jnp.int32),
        pltpu.VMEM((CHUNK, COLS), jnp.float32),
        pltpu.SemaphoreType.DMA,
    ],
    compiler_params=pltpu.CompilerParams(use_tc_tiling_on_sc=False),
)(_body)


@jax.jit
def kernel(prefix, emb_table):
    idx = prefix.reshape(NUM_ROWS // CHUNK, CHUNK).astype(jnp.int32)
    out = _gather(idx, emb_table)
    return out.reshape(BATCH, PRE_SEQ_LEN, OUT_DIM)
